# Initial kernel scaffold; baseline (speedup 1.0000x reference)
import jax
import jax.numpy as jnp
from jax.experimental import pallas as pl


def kernel(x, idx, emb1, emb2, lin1_w, lin1_b, lin2_w, lin2_b):
    raise NotImplementedError("write your pallas kernel here")


# speedup vs baseline: 8.6887x; 8.6887x over previous
import functools

import jax
import jax.numpy as jnp
from jax.experimental import pallas as pl
import jax.experimental.pallas.tpu as pltpu

N = 2048
DIM = 32
K = 30
L = 2
BLK_R = 256


def _mlp_kernel(emb1_ref, emb2_ref, w1_ref, b1_ref, w2_ref, b2_ref,
                nv1_ref, nv2_ref):
    nv1 = emb1_ref[...]
    nv2 = emb2_ref[...]
    for i in range(L):
        w1 = w1_ref[i]
        w2 = w2_ref[i]
        nv1 = jnp.tanh(
            jnp.dot(nv1, w1.T, preferred_element_type=jnp.float32) + b1_ref[i])
        nv2 = jnp.tanh(
            jnp.dot(nv2, w2.T, preferred_element_type=jnp.float32) + b2_ref[i])
    nv1_ref[...] = nv1
    nv2_ref[...] = nv2


def _main_kernel(xr_ref, xf_ref, nv1r_ref, nv2r_ref, nv1f_ref, nv2f_ref,
                 out_ref, acs_ref):
    b = pl.program_id(1)

    row0 = pl.program_id(0) * BLK_R
    col = jax.lax.broadcasted_iota(jnp.int32, (BLK_R, N), 1)
    row = row0 + jax.lax.broadcasted_iota(jnp.int32, (BLK_R, N), 0)
    diag = col == row

    @pl.when(b == 0)
    def _():
        a = (jnp.dot(nv1r_ref[...], nv2f_ref[...].T,
                     preferred_element_type=jnp.float32)
             - jnp.dot(nv2r_ref[...], nv1f_ref[...].T,
                       preferred_element_type=jnp.float32))
        acs_ref[...] = jax.nn.relu(jnp.tanh(a))

    s = jnp.dot(xr_ref[0], xf_ref[0].T, preferred_element_type=jnp.float32)
    s = jnp.where(diag, s - 1e8, s)
    s = jnp.where(s >= 0, s, 0.01 * s)
    m = jnp.max(s, axis=1, keepdims=True)
    e = jnp.exp(s - m)
    p = e / jnp.sum(e, axis=1, keepdims=True)
    astat = jnp.tanh(jnp.where(diag, p + 1.0, p))
    adj0 = acs_ref[...] + astat

    ai = jax.lax.bitcast_convert_type(adj0, jnp.int32)
    lo = jnp.zeros((BLK_R, 1), jnp.int32)
    hi = jnp.full((BLK_R, 1), 1 << 30, jnp.int32)
    for _ in range(31):
        mid = lo + ((hi - lo + 1) >> 1)
        cnt = jnp.sum((ai >= mid).astype(jnp.int32), axis=1, keepdims=True)
        ok = cnt >= K
        lo = jnp.where(ok, mid, lo)
        hi = jnp.where(ok, hi, mid - 1)
    gt = ai > lo
    tie = ai == lo
    need = K - jnp.sum(gt.astype(jnp.int32), axis=1, keepdims=True)
    tie_i = tie.astype(jnp.int32)
    tie_rank = jnp.concatenate(
        [jnp.zeros((BLK_R, 1), jnp.int32), tie_i[:, :-1]], axis=1)
    sh = 1
    while sh < N:
        tie_rank = tie_rank + jnp.concatenate(
            [jnp.zeros((BLK_R, sh), jnp.int32), tie_rank[:, :-sh]], axis=1)
        sh *= 2
    keep = gt | (tie & (tie_rank < need))
    out_ref[0] = jnp.where(keep, adj0, 0.0)


def kernel(x, idx, emb1, emb2, lin1_w, lin1_b, lin2_w, lin2_b):
    del idx
    bs = x.shape[0]

    nv1, nv2 = pl.pallas_call(
        _mlp_kernel,
        out_shape=[jax.ShapeDtypeStruct((N, DIM), jnp.float32)] * 2,
    )(emb1, emb2, lin1_w, lin1_b, lin2_w, lin2_b)

    grid = (N // BLK_R, bs)
    out = pl.pallas_call(
        _main_kernel,
        grid=grid,
        in_specs=[
            pl.BlockSpec((1, BLK_R, x.shape[2]), lambda r, b: (b, r, 0)),
            pl.BlockSpec((1, N, x.shape[2]), lambda r, b: (b, 0, 0)),
            pl.BlockSpec((BLK_R, DIM), lambda r, b: (r, 0)),
            pl.BlockSpec((BLK_R, DIM), lambda r, b: (r, 0)),
            pl.BlockSpec((N, DIM), lambda r, b: (0, 0)),
            pl.BlockSpec((N, DIM), lambda r, b: (0, 0)),
        ],
        out_specs=pl.BlockSpec((1, BLK_R, N), lambda r, b: (b, r, 0)),
        out_shape=jax.ShapeDtypeStruct((bs, N, N), jnp.float32),
        scratch_shapes=[pltpu.VMEM((BLK_R, N), jnp.float32)],
        compiler_params=pltpu.CompilerParams(
            dimension_semantics=("arbitrary", "arbitrary"),
        ),
    )(x, x, nv1, nv2, nv1, nv2)
    return out


# speedup vs baseline: 12.5397x; 1.4432x over previous
import jax
import jax.numpy as jnp
from jax.experimental import pallas as pl
import jax.experimental.pallas.tpu as pltpu

N = 2048
DIM = 32
K = 30
L = 2
BLK_R = 256


def _main_kernel(xr_ref, xf_ref, emb1_ref, emb2_ref, w1_ref, b1_ref,
                 w2_ref, b2_ref, out_ref, acs_ref, nv1_ref, nv2_ref):
    r = pl.program_id(0)
    b = pl.program_id(1)

    @pl.when((r == 0) & (b == 0))
    def _():
        nv1 = emb1_ref[...]
        nv2 = emb2_ref[...]
        for i in range(L):
            nv1 = jnp.tanh(
                jnp.dot(nv1, w1_ref[i].T, preferred_element_type=jnp.float32)
                + b1_ref[i])
            nv2 = jnp.tanh(
                jnp.dot(nv2, w2_ref[i].T, preferred_element_type=jnp.float32)
                + b2_ref[i])
        nv1_ref[...] = nv1
        nv2_ref[...] = nv2

    row0 = r * BLK_R
    col = jax.lax.broadcasted_iota(jnp.int32, (BLK_R, N), 1)
    row = row0 + jax.lax.broadcasted_iota(jnp.int32, (BLK_R, N), 0)
    diag = col == row

    @pl.when(b == 0)
    def _():
        nv1r = nv1_ref[pl.ds(row0, BLK_R), :]
        nv2r = nv2_ref[pl.ds(row0, BLK_R), :]
        a = (jnp.dot(nv1r, nv2_ref[...].T,
                     preferred_element_type=jnp.float32)
             - jnp.dot(nv2r, nv1_ref[...].T,
                       preferred_element_type=jnp.float32))
        acs_ref[...] = jax.nn.relu(jnp.tanh(a))

    s = jnp.dot(xr_ref[0], xf_ref[0].T, preferred_element_type=jnp.float32)
    s = jnp.where(diag, s - 1e8, s)
    s = jnp.where(s >= 0, s, 0.01 * s)
    m = jnp.max(s, axis=1, keepdims=True)
    e = jnp.exp(s - m)
    p = e / jnp.sum(e, axis=1, keepdims=True)
    astat = jnp.tanh(jnp.where(diag, p + 1.0, p))
    adj0 = acs_ref[...] + astat

    ai = jax.lax.bitcast_convert_type(adj0, jnp.int32)
    lo = jnp.zeros((BLK_R, 1), jnp.int32)
    hi = jnp.full((BLK_R, 1), 1 << 30, jnp.int32)
    cnt_gt = jnp.zeros((BLK_R, 1), jnp.int32)
    for _ in range(30):
        mid = lo + ((hi - lo + 1) >> 1)
        cnt = jnp.sum((ai >= mid).astype(jnp.int32), axis=1, keepdims=True)
        ok = cnt >= K
        lo = jnp.where(ok, mid, lo)
        hi = jnp.where(ok, hi, mid - 1)
        cnt_gt = jnp.where(ok, cnt_gt, cnt)
    gt = ai > lo
    tie = ai == lo
    need = (K - cnt_gt).astype(jnp.float32)
    tie_f = jnp.where(tie, 1.0, 0.0).astype(jnp.float32)
    C = 128
    tri = (jax.lax.broadcasted_iota(jnp.int32, (C, C), 0)
           < jax.lax.broadcasted_iota(jnp.int32, (C, C), 1)).astype(jnp.float32)
    carry = jnp.zeros((BLK_R, 1), jnp.float32)
    for c in range(N // C):
        tf_c = tie_f[:, c * C:(c + 1) * C]
        ex = jnp.dot(tf_c, tri, preferred_element_type=jnp.float32)
        rank = ex + carry
        keep = gt[:, c * C:(c + 1) * C] | (
            tie[:, c * C:(c + 1) * C] & (rank < need))
        out_ref[0, :, c * C:(c + 1) * C] = jnp.where(
            keep, adj0[:, c * C:(c + 1) * C], 0.0)
        carry = carry + ex[:, C - 1:C] + tf_c[:, C - 1:C]


def kernel(x, idx, emb1, emb2, lin1_w, lin1_b, lin2_w, lin2_b):
    del idx
    bs = x.shape[0]
    feat = x.shape[2]

    grid = (N // BLK_R, bs)
    out = pl.pallas_call(
        _main_kernel,
        grid=grid,
        in_specs=[
            pl.BlockSpec((1, BLK_R, feat), lambda r, b: (b, r, 0)),
            pl.BlockSpec((1, N, feat), lambda r, b: (b, 0, 0)),
            pl.BlockSpec((N, DIM), lambda r, b: (0, 0)),
            pl.BlockSpec((N, DIM), lambda r, b: (0, 0)),
            pl.BlockSpec((L, DIM, DIM), lambda r, b: (0, 0, 0)),
            pl.BlockSpec((L, DIM), lambda r, b: (0, 0)),
            pl.BlockSpec((L, DIM, DIM), lambda r, b: (0, 0, 0)),
            pl.BlockSpec((L, DIM), lambda r, b: (0, 0)),
        ],
        out_specs=pl.BlockSpec((1, BLK_R, N), lambda r, b: (b, r, 0)),
        out_shape=jax.ShapeDtypeStruct((bs, N, N), jnp.float32),
        scratch_shapes=[
            pltpu.VMEM((BLK_R, N), jnp.float32),
            pltpu.VMEM((N, DIM), jnp.float32),
            pltpu.VMEM((N, DIM), jnp.float32),
        ],
        compiler_params=pltpu.CompilerParams(
            dimension_semantics=("arbitrary", "arbitrary"),
        ),
    )(x, x, emb1, emb2, lin1_w, lin1_b, lin2_w, lin2_b)
    return out


# speedup vs baseline: 12.6653x; 1.0100x over previous
import jax
import jax.numpy as jnp
from jax.experimental import pallas as pl
import jax.experimental.pallas.tpu as pltpu

N = 2048
DIM = 32
K = 30
L = 2
BLK_R = 512


def _main_kernel(xr_ref, xf_ref, emb1_ref, emb2_ref, w1_ref, b1_ref,
                 w2_ref, b2_ref, out_ref, acs_ref, diag_ref, nv1_ref,
                 nv2_ref):
    r = pl.program_id(0)
    b = pl.program_id(1)

    @pl.when((r == 0) & (b == 0))
    def _():
        nv1 = emb1_ref[...]
        nv2 = emb2_ref[...]
        for i in range(L):
            nv1 = jnp.tanh(
                jnp.dot(nv1, w1_ref[i].T, preferred_element_type=jnp.float32)
                + b1_ref[i])
            nv2 = jnp.tanh(
                jnp.dot(nv2, w2_ref[i].T, preferred_element_type=jnp.float32)
                + b2_ref[i])
        nv1_ref[...] = nv1
        nv2_ref[...] = nv2

    row0 = r * BLK_R

    @pl.when(b == 0)
    def _():
        nv1r = nv1_ref[pl.ds(row0, BLK_R), :]
        nv2r = nv2_ref[pl.ds(row0, BLK_R), :]
        a = (jnp.dot(nv1r, nv2_ref[...].T,
                     preferred_element_type=jnp.float32)
             - jnp.dot(nv2r, nv1_ref[...].T,
                       preferred_element_type=jnp.float32))
        acs_ref[...] = jax.nn.relu(jnp.tanh(a))
        col = jax.lax.broadcasted_iota(jnp.int32, (BLK_R, N), 1)
        rw = row0 + jax.lax.broadcasted_iota(jnp.int32, (BLK_R, N), 0)
        diag_ref[...] = jnp.where(col == rw, 1.0, 0.0)

    diag_f = diag_ref[...]
    s = jnp.dot(xr_ref[0], xf_ref[0].T, preferred_element_type=jnp.float32)
    s = s - diag_f * 1e8
    s = jnp.where(s >= 0, s, 0.01 * s)
    m = jnp.max(s, axis=1, keepdims=True)
    e = jnp.exp(s - m)
    p = e / jnp.sum(e, axis=1, keepdims=True)
    astat = jnp.tanh(p + diag_f)
    adj0 = acs_ref[...] + astat

    ai = jax.lax.bitcast_convert_type(adj0, jnp.int32)
    lo = jnp.zeros((BLK_R, 1), jnp.int32)
    hi = jnp.full((BLK_R, 1), 1 << 30, jnp.int32)
    cnt_gt = jnp.zeros((BLK_R, 1), jnp.int32)
    for _ in range(30):
        mid = lo + ((hi - lo + 1) >> 1)
        cnt = jnp.sum((ai >= mid).astype(jnp.int32), axis=1, keepdims=True)
        ok = cnt >= K
        lo = jnp.where(ok, mid, lo)
        hi = jnp.where(ok, hi, mid - 1)
        cnt_gt = jnp.where(ok, cnt_gt, cnt)
    gt = ai > lo
    tie = ai == lo
    need = (K - cnt_gt).astype(jnp.float32)
    tie_f = jnp.where(tie, 1.0, 0.0).astype(jnp.float32)
    C = 128
    tri = (jax.lax.broadcasted_iota(jnp.int32, (C, C), 0)
           < jax.lax.broadcasted_iota(jnp.int32, (C, C), 1)).astype(jnp.float32)
    carry = jnp.zeros((BLK_R, 1), jnp.float32)
    for c in range(N // C):
        tf_c = tie_f[:, c * C:(c + 1) * C]
        ex = jnp.dot(tf_c, tri, preferred_element_type=jnp.float32)
        rank = ex + carry
        keep = gt[:, c * C:(c + 1) * C] | (
            tie[:, c * C:(c + 1) * C] & (rank < need))
        out_ref[0, :, c * C:(c + 1) * C] = jnp.where(
            keep, adj0[:, c * C:(c + 1) * C], 0.0)
        carry = carry + ex[:, C - 1:C] + tf_c[:, C - 1:C]


def kernel(x, idx, emb1, emb2, lin1_w, lin1_b, lin2_w, lin2_b):
    del idx
    bs = x.shape[0]
    feat = x.shape[2]

    grid = (N // BLK_R, bs)
    out = pl.pallas_call(
        _main_kernel,
        grid=grid,
        in_specs=[
            pl.BlockSpec((1, BLK_R, feat), lambda r, b: (b, r, 0)),
            pl.BlockSpec((1, N, feat), lambda r, b: (b, 0, 0)),
            pl.BlockSpec((N, DIM), lambda r, b: (0, 0)),
            pl.BlockSpec((N, DIM), lambda r, b: (0, 0)),
            pl.BlockSpec((L, DIM, DIM), lambda r, b: (0, 0, 0)),
            pl.BlockSpec((L, DIM), lambda r, b: (0, 0)),
            pl.BlockSpec((L, DIM, DIM), lambda r, b: (0, 0, 0)),
            pl.BlockSpec((L, DIM), lambda r, b: (0, 0)),
        ],
        out_specs=pl.BlockSpec((1, BLK_R, N), lambda r, b: (b, r, 0)),
        out_shape=jax.ShapeDtypeStruct((bs, N, N), jnp.float32),
        scratch_shapes=[
            pltpu.VMEM((BLK_R, N), jnp.float32),
            pltpu.VMEM((BLK_R, N), jnp.float32),
            pltpu.VMEM((N, DIM), jnp.float32),
            pltpu.VMEM((N, DIM), jnp.float32),
        ],
        compiler_params=pltpu.CompilerParams(
            dimension_semantics=("arbitrary", "arbitrary"),
        ),
    )(x, x, emb1, emb2, lin1_w, lin1_b, lin2_w, lin2_b)
    return out


# speedup vs baseline: 12.6747x; 1.0007x over previous
import jax
import jax.numpy as jnp
from jax.experimental import pallas as pl
import jax.experimental.pallas.tpu as pltpu

N = 2048
DIM = 32
K = 30
L = 2
BLK_R = 512


def _main_kernel(xr_ref, xf_ref, emb1_ref, emb2_ref, w1_ref, b1_ref,
                 w2_ref, b2_ref, out_ref, acs_ref, diag_ref, nv1_ref,
                 nv2_ref):
    r = pl.program_id(0)
    b = pl.program_id(1)

    @pl.when((r == 0) & (b == 0))
    def _():
        nv1 = emb1_ref[...]
        nv2 = emb2_ref[...]
        for i in range(L):
            nv1 = jnp.tanh(
                jnp.dot(nv1, w1_ref[i].T, preferred_element_type=jnp.float32)
                + b1_ref[i])
            nv2 = jnp.tanh(
                jnp.dot(nv2, w2_ref[i].T, preferred_element_type=jnp.float32)
                + b2_ref[i])
        nv1_ref[...] = nv1
        nv2_ref[...] = nv2

    row0 = r * BLK_R

    @pl.when(b == 0)
    def _():
        nv1r = nv1_ref[pl.ds(row0, BLK_R), :]
        nv2r = nv2_ref[pl.ds(row0, BLK_R), :]
        a = (jnp.dot(nv1r, nv2_ref[...].T,
                     preferred_element_type=jnp.float32)
             - jnp.dot(nv2r, nv1_ref[...].T,
                       preferred_element_type=jnp.float32))
        acs_ref[...] = jax.nn.relu(jnp.tanh(a))
        col = jax.lax.broadcasted_iota(jnp.int32, (BLK_R, N), 1)
        rw = row0 + jax.lax.broadcasted_iota(jnp.int32, (BLK_R, N), 0)
        diag_ref[...] = jnp.where(col == rw, 1.0, 0.0)

    diag_f = diag_ref[...]
    s = jnp.dot(xr_ref[0], xf_ref[0].T, preferred_element_type=jnp.float32)
    s = s - diag_f * 1e8
    s = jnp.where(s >= 0, s, 0.01 * s)
    m = jnp.max(s, axis=1, keepdims=True)
    e = jnp.exp(s - m)
    p = e / jnp.sum(e, axis=1, keepdims=True)
    astat = jnp.tanh(p + diag_f)
    adj0 = acs_ref[...] + astat

    ai = jax.lax.bitcast_convert_type(adj0, jnp.int32)
    lo = jnp.zeros((BLK_R, 1), jnp.int32)
    hi = jnp.full((BLK_R, 1), (1 << 30) - 1, jnp.int32)
    cnt_gt = jnp.zeros((BLK_R, 1), jnp.int32)
    for _ in range(30):
        mid = lo + ((hi - lo + 1) >> 1)
        cnt = jnp.sum((ai >= mid).astype(jnp.int32), axis=1, keepdims=True)
        ok = cnt >= K
        lo = jnp.where(ok, mid, lo)
        hi = jnp.where(ok, hi, mid - 1)
        cnt_gt = jnp.where(ok, cnt_gt, cnt)
    gt = ai > lo
    tie = ai == lo
    need = (K - cnt_gt).astype(jnp.float32)
    tie_f = jnp.where(tie, 1.0, 0.0).astype(jnp.float32)
    C = 128
    tri = (jax.lax.broadcasted_iota(jnp.int32, (C, C), 0)
           < jax.lax.broadcasted_iota(jnp.int32, (C, C), 1)).astype(jnp.float32)
    carry = jnp.zeros((BLK_R, 1), jnp.float32)
    for c in range(N // C):
        tf_c = tie_f[:, c * C:(c + 1) * C]
        ex = jnp.dot(tf_c, tri, preferred_element_type=jnp.float32)
        rank = ex + carry
        keep = gt[:, c * C:(c + 1) * C] | (
            tie[:, c * C:(c + 1) * C] & (rank < need))
        out_ref[0, :, c * C:(c + 1) * C] = jnp.where(
            keep, adj0[:, c * C:(c + 1) * C], 0.0)
        carry = carry + ex[:, C - 1:C] + tf_c[:, C - 1:C]


def kernel(x, idx, emb1, emb2, lin1_w, lin1_b, lin2_w, lin2_b):
    del idx
    bs = x.shape[0]
    feat = x.shape[2]

    grid = (N // BLK_R, bs)
    out = pl.pallas_call(
        _main_kernel,
        grid=grid,
        in_specs=[
            pl.BlockSpec((1, BLK_R, feat), lambda r, b: (b, r, 0)),
            pl.BlockSpec((1, N, feat), lambda r, b: (b, 0, 0)),
            pl.BlockSpec((N, DIM), lambda r, b: (0, 0)),
            pl.BlockSpec((N, DIM), lambda r, b: (0, 0)),
            pl.BlockSpec((L, DIM, DIM), lambda r, b: (0, 0, 0)),
            pl.BlockSpec((L, DIM), lambda r, b: (0, 0)),
            pl.BlockSpec((L, DIM, DIM), lambda r, b: (0, 0, 0)),
            pl.BlockSpec((L, DIM), lambda r, b: (0, 0)),
        ],
        out_specs=pl.BlockSpec((1, BLK_R, N), lambda r, b: (b, r, 0)),
        out_shape=jax.ShapeDtypeStruct((bs, N, N), jnp.float32),
        scratch_shapes=[
            pltpu.VMEM((BLK_R, N), jnp.float32),
            pltpu.VMEM((BLK_R, N), jnp.float32),
            pltpu.VMEM((N, DIM), jnp.float32),
            pltpu.VMEM((N, DIM), jnp.float32),
        ],
        compiler_params=pltpu.CompilerParams(
            dimension_semantics=("arbitrary", "arbitrary"),
        ),
    )(x, x, emb1, emb2, lin1_w, lin1_b, lin2_w, lin2_b)
    return out
